# Initial kernel scaffold; baseline (speedup 1.0000x reference)
import jax
import jax.numpy as jnp
from jax.experimental import pallas as pl


def kernel(user_emb, entity_emb, edge_index, edge_type, inter_edge, inter_edge_w, relation_emb, W_Q):
    raise NotImplementedError("write your pallas kernel here")


# speedup vs baseline: 2.2295x; 2.2295x over previous
import functools

import jax
import jax.numpy as jnp
from jax import lax
from jax.experimental import pallas as pl
from jax.experimental.pallas import tpu as pltpu
from jax.experimental.pallas import tpu_sc as plsc

_L = 16
_NS = 16
_CHUNK = 32
_D = 128
_DEN_ROWS = 160


def _mm_body(e_ref, w_ref, o_ref):
    o_ref[...] = jnp.dot(e_ref[...], w_ref[...], preferred_element_type=jnp.float32)


def _compute_q(emb, w):
    n = emb.shape[0]
    blk = 1000
    return pl.pallas_call(
        _mm_body,
        grid=(n // blk,),
        in_specs=[pl.BlockSpec((blk, _D), lambda i: (i, 0)),
                  pl.BlockSpec((_D, _D), lambda i: (0, 0))],
        out_specs=pl.BlockSpec((blk, _D), lambda i: (i, 0)),
        out_shape=jax.ShapeDtypeStruct((n, _D), jnp.float32),
    )(emb, w)


def _combine_body(acc_e_ref, den_ref, acc_u_ref, eres_ref, ures_ref,
                  enext_ref, ereso_ref, ureso_ref):
    acc = acc_e_ref[...]
    rows = acc.shape[0]
    cols = lax.broadcasted_iota(jnp.int32, (rows, _D), 1)
    den = jnp.where(cols < 64, den_ref[:, 0:1], den_ref[:, 1:2]) + 1e-16
    agg = acc / den
    n = jnp.sqrt(jnp.sum(agg * agg, axis=1, keepdims=True))
    e_next = agg / jnp.maximum(n, 1e-12)
    u = acc_u_ref[...]
    nu = jnp.sqrt(jnp.sum(u * u, axis=1, keepdims=True))
    u_next = u / jnp.maximum(nu, 1e-12)
    enext_ref[...] = e_next
    ereso_ref[...] = eres_ref[...] + e_next
    ureso_ref[...] = ures_ref[...] + u_next


def _combine(acc_e, den2, acc_u, e_res, u_res):
    n = e_res.shape[0]
    blk = 1000
    specs = dict(
        grid=(n // blk,),
        in_specs=[pl.BlockSpec((blk, _D), lambda i: (i, 0)),
                  pl.BlockSpec((blk, 2), lambda i: (i, 0)),
                  pl.BlockSpec((blk, _D), lambda i: (i, 0)),
                  pl.BlockSpec((blk, _D), lambda i: (i, 0)),
                  pl.BlockSpec((blk, _D), lambda i: (i, 0))],
        out_specs=[pl.BlockSpec((blk, _D), lambda i: (i, 0))] * 3,
        out_shape=[jax.ShapeDtypeStruct((n, _D), jnp.float32)] * 3,
    )
    return pl.pallas_call(_combine_body, **specs)(acc_e, den2, acc_u, e_res,
                                                  u_res)


@functools.lru_cache(maxsize=None)
def _make_sc_hop(n_pad, e_total, eui_total):
    mesh = plsc.VectorSubcoreMesh(core_axis_name="c", subcore_axis_name="s")
    e_chunks = e_total // _CHUNK
    u_chunks = eui_total // _CHUNK
    n_acc = n_pad + _DEN_ROWS
    zrows = n_pad // _NS
    grp = _CHUNK // _L

    @functools.partial(
        pl.kernel,
        mesh=mesh,
        out_type=(jax.ShapeDtypeStruct((n_acc, _D), jnp.float32),
                  jax.ShapeDtypeStruct((n_acc, _D), jnp.float32)),
        scratch_types=[
            pltpu.VMEM_SHARED((n_acc, _D), jnp.float32),
            pltpu.VMEM((_CHUNK,), jnp.int32),
            pltpu.VMEM((_CHUNK,), jnp.int32),
            pltpu.VMEM((_CHUNK,), jnp.int32),
            pltpu.VMEM((_CHUNK,), jnp.int32),
            pltpu.VMEM((_CHUNK,), jnp.float32),
            pltpu.VMEM((_CHUNK, _D), jnp.float32),
            pltpu.VMEM((_CHUNK, _D), jnp.float32),
            pltpu.VMEM((_CHUNK, _D), jnp.float32),
            pltpu.VMEM((_CHUNK, _D), jnp.float32),
            pltpu.VMEM((_CHUNK, _D), jnp.float32),
            pltpu.VMEM((_CHUNK, _D), jnp.float32),
        ],
        compiler_params=pltpu.CompilerParams(needs_layout_passes=False),
    )
    def sc_hop(zeros_hbm, qtab, embtab, r2tab, headv, tailv, etypev, iuserv,
               iitemv, iwv, out_ev, out_uv, acc, hidx, tidx, ridx, hgidx,
               wbuf, qh, qt, et, rel, contrib, dbuf):
        core = lax.axis_index("c")
        s = lax.axis_index("s")
        lanes = lax.iota(jnp.int32, _L)
        zero16 = jnp.zeros((_L,), jnp.float32)

        zbase = s * zrows

        def _zacc(k, _):
            pltpu.sync_copy(zeros_hbm,
                            acc.at[pl.ds(zbase + k * _CHUNK, _CHUNK)])
            return 0
        lax.fori_loop(0, zrows // _CHUNK, _zacc, 0)

        @pl.when(s == 0)
        def _zden():
            def _zd(k, _):
                pltpu.sync_copy(zeros_hbm,
                                acc.at[pl.ds(n_pad + k * _CHUNK, _CHUNK)])
                return 0
            lax.fori_loop(0, _DEN_ROWS // _CHUNK, _zd, 0)
        plsc.subcore_barrier()

        @pl.when(core == 0)
        def _edge_pass():
            def chunk_body(i, _):
                base = (s + i * _NS) * _CHUNK
                pltpu.sync_copy(headv.at[pl.ds(base, _CHUNK)], hidx)
                pltpu.sync_copy(tailv.at[pl.ds(base, _CHUNK)], tidx)
                pltpu.sync_copy(etypev.at[pl.ds(base, _CHUNK)], ridx)
                pltpu.sync_copy(qtab.at[hidx], qh)
                pltpu.sync_copy(qtab.at[tidx], qt)
                pltpu.sync_copy(embtab.at[tidx], et)
                pltpu.sync_copy(r2tab.at[ridx], rel)

                def grp_body(g, _):
                    hv = hidx[pl.ds(g * _L, _L)]
                    hgidx[pl.ds(g * _L, _L)] = \
                        lax.shift_right_logical(hv, 6) + n_pad
                    for lane in range(_L):
                        e = g * _L + lane
                        s0 = zero16
                        s1 = zero16
                        for cb in range(4):
                            sl = pl.ds(cb * _L, _L)
                            s0 = s0 + qh[e, sl] * qt[e, sl] * rel[e, sl]
                        for cb in range(4, 8):
                            sl = pl.ds(cb * _L, _L)
                            s1 = s1 + qh[e, sl] * qt[e, sl] * rel[e, sl]
                        t0 = jnp.sum(s0) * 0.125
                        t1 = jnp.sum(s1) * 0.125
                        ex0 = jnp.exp(jnp.full((_L,), t0))
                        ex1 = jnp.exp(jnp.full((_L,), t1))
                        for cb in range(4):
                            sl = pl.ds(cb * _L, _L)
                            contrib[e, sl] = et[e, sl] * rel[e, sl] * ex0
                        for cb in range(4, 8):
                            sl = pl.ds(cb * _L, _L)
                            contrib[e, sl] = et[e, sl] * rel[e, sl] * ex1
                        h = hv[lane]
                        c2 = (h % 64) * 2
                        cb_t = c2 // _L
                        cl = c2 % _L
                        exslice = jnp.where(lanes == cl, ex0,
                                            jnp.where(lanes == cl + 1, ex1,
                                                      zero16))
                        for cb in range(_D // _L):
                            dbuf[e, pl.ds(cb * _L, _L)] = \
                                jnp.where(cb_t == cb, exslice, zero16)
                    return 0
                lax.fori_loop(0, grp, grp_body, 0)
                pltpu.sync_copy(contrib, acc.at[hidx], add=True)
                pltpu.sync_copy(dbuf, acc.at[hgidx], add=True)
                return 0
            lax.fori_loop(0, e_chunks // _NS, chunk_body, 0)

        @pl.when(core == 1)
        def _user_pass():
            nc = jnp.where(s < u_chunks % _NS,
                           u_chunks // _NS + 1, u_chunks // _NS)

            def chunk_body(i, _):
                base = (s + i * _NS) * _CHUNK
                pltpu.sync_copy(iuserv.at[pl.ds(base, _CHUNK)], hidx)
                pltpu.sync_copy(iitemv.at[pl.ds(base, _CHUNK)], tidx)
                pltpu.sync_copy(iwv.at[pl.ds(base, _CHUNK)], wbuf)
                pltpu.sync_copy(embtab.at[tidx], qh)

                def grp_body(g, _):
                    wv = wbuf[pl.ds(g * _L, _L)]
                    for lane in range(_L):
                        e = g * _L + lane
                        wb = jnp.full((_L,), wv[lane])
                        for cb in range(8):
                            sl = pl.ds(cb * _L, _L)
                            contrib[e, sl] = qh[e, sl] * wb
                    return 0
                lax.fori_loop(0, grp, grp_body, 0)
                pltpu.sync_copy(contrib, acc.at[hidx], add=True)
                return 0
            lax.fori_loop(0, nc, chunk_body, 0)

        plsc.subcore_barrier()

        out = [out_ev, out_uv]
        for ci in range(2):
            @pl.when(core == ci)
            def _out(o=out[ci]):
                def _cp(k, _):
                    sl = pl.ds(zbase + k * 80, 80)
                    pltpu.sync_copy(acc.at[sl], o.at[sl])
                    return 0
                lax.fori_loop(0, zrows // 80, _cp, 0)

                @pl.when(s == 0)
                def _cpden():
                    def _cpd(k, _):
                        sl = pl.ds(n_pad + k * 80, 80)
                        pltpu.sync_copy(acc.at[sl], o.at[sl])
                        return 0
                    lax.fori_loop(0, _DEN_ROWS // 80, _cpd, 0)

    return sc_hop


def kernel(user_emb, entity_emb, edge_index, edge_type, inter_edge,
           inter_edge_w, relation_emb, W_Q):
    n_ent = entity_emb.shape[0]
    n_usr = user_emb.shape[0]
    e_total = edge_index.shape[1]
    eui_total = inter_edge.shape[1]
    n_pad = ((max(n_ent, n_usr) + _NS * 80 - 1) // (_NS * 80)) * (_NS * 80)

    head = edge_index[0]
    tail = edge_index[1]
    iu = inter_edge[0]
    ii = inter_edge[1]
    r2 = jnp.roll(relation_emb, 1, axis=0)

    sc_hop = _make_sc_hop(n_pad, e_total, eui_total)
    zpad = jnp.zeros((_CHUNK, _D), jnp.float32)

    e_res = entity_emb
    u_res = user_emb
    emb = entity_emb
    for _ in range(2):
        q = _compute_q(emb, W_Q)
        oe, ou = sc_hop(zpad, q, emb, r2, head, tail, edge_type, iu, ii,
                        inter_edge_w)
        den2 = oe[n_pad:].reshape(-1, 2)[:n_ent]
        emb, e_res, u_res = _combine(oe[:n_ent], den2, ou[:n_usr],
                                     e_res, u_res)
    return (e_res, u_res)
